# initial kernel scaffold (unmeasured)
import jax
import jax.numpy as jnp
from jax import lax
from jax.experimental import pallas as pl
from jax.experimental.pallas import tpu as pltpu

N_DEV = 16
M = 8192
K = 8192
N = 4096
M_PER = M // N_DEV
N_PER = N // N_DEV

_C = 0.7978845608028654


def _gelu(y):
    return 0.5 * y * (1.0 + jnp.tanh(_C * (y + 0.044715 * y * y * y)))


def kernel(x, w_mat):
    def body(x_ref, w_hbm, out_ref, w_buf, y_buf, load_sems, send_sems, recv_sems):
        i = lax.axis_index("i")

        def mk_load(j):
            d = (i + j) % N_DEV
            return pltpu.make_async_copy(
                w_hbm.at[:, pl.ds(d * N_PER, N_PER)],
                w_buf.at[j % 2],
                load_sems.at[j % 2],
            )

        loads = [None] * N_DEV
        loads[0] = mk_load(0)
        loads[0].start()

        xv = x_ref[...]
        pending_sends = {}
        for j in range(N_DEV):
            if j + 1 < N_DEV:
                loads[j + 1] = mk_load(j + 1)
                loads[j + 1].start()
            loads[j].wait()

            d = (i + j) % N_DEV
            y = _gelu(jnp.dot(xv, w_buf[j % 2], preferred_element_type=jnp.float32))
            if j == 0:
                out_ref[pl.ds(i * M_PER, M_PER), :] = y
            else:
                sslot = j % 2
                if j - 2 in pending_sends:
                    pending_sends.pop(j - 2).wait_send()
                y_buf[sslot, :, :] = y
                rdma = pltpu.make_async_remote_copy(
                    src_ref=y_buf.at[sslot],
                    dst_ref=out_ref.at[pl.ds(i * M_PER, M_PER), :],
                    send_sem=send_sems.at[j],
                    recv_sem=recv_sems.at[j],
                    device_id=(d,),
                    device_id_type=pl.DeviceIdType.MESH,
                )
                rdma.start()
                pending_sends[j] = rdma

        for rdma in pending_sends.values():
            rdma.wait_send()

        for j in range(1, N_DEV):
            s = (i - j + N_DEV) % N_DEV
            recv = pltpu.make_async_remote_copy(
                src_ref=y_buf.at[0],
                dst_ref=out_ref.at[pl.ds(s * M_PER, M_PER), :],
                send_sem=send_sems.at[j],
                recv_sem=recv_sems.at[j],
                device_id=(i,),
                device_id_type=pl.DeviceIdType.MESH,
            )
            recv.wait_recv()

    return pl.pallas_call(
        body,
        out_shape=jax.ShapeDtypeStruct((M, N_PER), jnp.float32),
        in_specs=[
            pl.BlockSpec(memory_space=pltpu.VMEM),
            pl.BlockSpec(memory_space=pltpu.ANY),
        ],
        out_specs=pl.BlockSpec(memory_space=pltpu.VMEM),
        scratch_shapes=[
            pltpu.VMEM((2, K, N_PER), jnp.float32),
            pltpu.VMEM((2, M_PER, N_PER), jnp.float32),
            pltpu.SemaphoreType.DMA((2,)),
            pltpu.SemaphoreType.DMA((N_DEV,)),
            pltpu.SemaphoreType.DMA((N_DEV,)),
        ],
        compiler_params=pltpu.CompilerParams(collective_id=0),
    )(x, w_mat)


# baseline (device time: 164788 ns/iter reference)
import jax
import jax.numpy as jnp
from jax import lax
from jax.experimental import pallas as pl
from jax.experimental.pallas import tpu as pltpu

N_DEV = 16
M = 8192
K = 8192
N = 4096
M_PER = M // N_DEV
N_PER = N // N_DEV

_C = 0.7978845608028654


def _gelu(y):
    return 0.5 * y * (1.0 + jnp.tanh(_C * (y + 0.044715 * y * y * y)))


def kernel(x, w_mat):
    def body(x_ref, w_hbm, out_ref, w_buf, y_buf, load_sems, send_sems, recv_sems):
        i = lax.axis_index("i")

        def mk_load(j):
            d = (i + j) % N_DEV
            return pltpu.make_async_copy(
                w_hbm.at[:, pl.ds(d * N_PER, N_PER)],
                w_buf.at[j % 2],
                load_sems.at[j % 2],
            )

        loads = [None] * N_DEV
        loads[0] = mk_load(0)
        loads[0].start()

        xv = x_ref[...]
        pending_sends = {}
        for j in range(N_DEV):
            if j + 1 < N_DEV:
                loads[j + 1] = mk_load(j + 1)
                loads[j + 1].start()
            loads[j].wait()

            d = (i + j) % N_DEV
            y = _gelu(jnp.dot(xv, w_buf[j % 2], preferred_element_type=jnp.float32))
            if j == 0:
                out_ref[pl.ds(i * M_PER, M_PER), :] = y
            else:
                sslot = j % 2
                if j - 2 in pending_sends:
                    pending_sends.pop(j - 2).wait_send()
                y_buf[sslot, :, :] = y
                rdma = pltpu.make_async_remote_copy(
                    src_ref=y_buf.at[sslot],
                    dst_ref=out_ref.at[pl.ds(i * M_PER, M_PER), :],
                    send_sem=send_sems.at[j],
                    recv_sem=recv_sems.at[j],
                    device_id=(d,),
                    device_id_type=pl.DeviceIdType.MESH,
                )
                rdma.start()
                pending_sends[j] = rdma

        for rdma in pending_sends.values():
            rdma.wait_send()

        for j in range(1, N_DEV):
            s = (i - j + N_DEV) % N_DEV
            recv = pltpu.make_async_remote_copy(
                src_ref=y_buf.at[0],
                dst_ref=out_ref.at[pl.ds(s * M_PER, M_PER), :],
                send_sem=send_sems.at[j],
                recv_sem=recv_sems.at[j],
                device_id=(i,),
                device_id_type=pl.DeviceIdType.MESH,
            )
            recv.wait_recv()

    return pl.pallas_call(
        body,
        out_shape=jax.ShapeDtypeStruct((M, N_PER), jnp.float32),
        in_specs=[
            pl.BlockSpec(memory_space=pltpu.VMEM),
            pl.BlockSpec(memory_space=pl.ANY),
        ],
        out_specs=pl.BlockSpec(memory_space=pltpu.VMEM),
        scratch_shapes=[
            pltpu.VMEM((2, K, N_PER), jnp.float32),
            pltpu.VMEM((2, M_PER, N_PER), jnp.float32),
            pltpu.SemaphoreType.DMA((2,)),
            pltpu.SemaphoreType.DMA((N_DEV,)),
            pltpu.SemaphoreType.DMA((N_DEV,)),
        ],
        compiler_params=pltpu.CompilerParams(
            vmem_limit_bytes=100 * 1024 * 1024,
        ),
    )(x, w_mat)
